# baseline (speedup 1.0000x reference)
import math

import jax
import jax.numpy as jnp
from jax import lax
from jax.experimental import pallas as pl
from jax.experimental.pallas import tpu as pltpu
from jax.experimental.pallas import tpu_sc as plsc

N = 10000
D = 128
E = 320000
NPAD = 10240
ALPHA = 0.5
BETA = math.log(2.0)
C1 = (1.0 - ALPHA) * (1.0 - BETA)
C2 = ALPHA * (1.0 - BETA)

NSC = 2
NTEC = 16

KA = 128
EPT_A = E // (NSC * NTEC)
NCHA = -(-EPT_A // KA)
EPT_A_PAD = NCHA * KA
ROWS_PER_TILE_A = NPAD // NTEC

KC = 128
GCH = 16
NGC = 5
G0 = 8
G1 = 2 * NGC - G0
NGRP = NSC * NTEC * NGC
EPT_C_PAD = NGC * GCH * KC
ROWS_PER_TILE_C = NPAD // NTEC

_mesh = plsc.VectorSubcoreMesh(core_axis_name="c", subcore_axis_name="s")


def _deg_body(col_hbm, out_hbm, hist_sh, idx_v, ones_v, zrow_v):
    c = lax.axis_index("c")
    s = lax.axis_index("s")
    t = c * NTEC + s
    base = s * ROWS_PER_TILE_A

    def fill_ones(i, carry):
        ones_v[pl.ds(i * 16, 16)] = jnp.ones((16,), jnp.float32)
        return carry

    lax.fori_loop(0, KA // 16, fill_ones, 0)

    def fill_zero(i, carry):
        zrow_v[pl.ds(i * 16, 16)] = jnp.zeros((16,), jnp.float32)
        return carry

    lax.fori_loop(0, ROWS_PER_TILE_A // 16, fill_zero, 0)

    pltpu.sync_copy(zrow_v, hist_sh.at[pl.ds(base, ROWS_PER_TILE_A)])
    pltpu.sync_copy(col_hbm.at[t], idx_v)
    plsc.subcore_barrier()

    def step(j, carry):
        pltpu.sync_copy(ones_v, hist_sh.at[idx_v.at[j]], add=True)
        return carry

    lax.fori_loop(0, NCHA, step, 0)
    plsc.subcore_barrier()
    pltpu.sync_copy(hist_sh.at[pl.ds(base, ROWS_PER_TILE_A)],
                    out_hbm.at[c, pl.ds(base, ROWS_PER_TILE_A)])


_deg_call = pl.kernel(
    _deg_body,
    out_type=jax.ShapeDtypeStruct((NSC, NPAD), jnp.float32),
    mesh=_mesh,
    scratch_types=[
        pltpu.VMEM_SHARED((NPAD,), jnp.float32),
        pltpu.VMEM((NCHA, KA), jnp.int32),
        pltpu.VMEM((KA,), jnp.float32),
        pltpu.VMEM((ROWS_PER_TILE_A,), jnp.float32),
    ],
)


def _scatter_body(y_hbm, row_hbm, col_hbm, out_hbm,
                  agg_sh, ridx_v, cidx_v, rows0_v, rows1_v, sem0, sem1):
    c = lax.axis_index("c")
    s = lax.axis_index("s")
    base = s * ROWS_PER_TILE_C

    def zrow(r, carry):
        def zlane(k, inner):
            rows0_v[r, pl.ds(k * 16, 16)] = jnp.zeros((16,), jnp.float32)
            return inner
        return lax.fori_loop(0, D // 16, zlane, carry)

    lax.fori_loop(0, KC, zrow, 0)

    def zcopy(k, carry):
        pltpu.sync_copy(rows0_v, agg_sh.at[pl.ds(base + k * KC, KC)])
        return carry

    lax.fori_loop(0, ROWS_PER_TILE_C // KC, zcopy, 0)
    plsc.subcore_barrier()

    HK = KC // 2

    def gat_start(j, buf, sem):
        pltpu.async_copy(y_hbm.at[ridx_v.at[j, pl.ds(0, HK)]],
                         buf.at[pl.ds(0, HK)], sem)
        pltpu.async_copy(y_hbm.at[ridx_v.at[j, pl.ds(HK, HK)]],
                         buf.at[pl.ds(HK, HK)], sem)

    def gat_wait(j, buf, sem):
        pltpu.make_async_copy(y_hbm.at[ridx_v.at[j, pl.ds(0, HK)]],
                              buf.at[pl.ds(0, HK)], sem).wait()
        pltpu.make_async_copy(y_hbm.at[ridx_v.at[j, pl.ds(HK, HK)]],
                              buf.at[pl.ds(HK, HK)], sem).wait()

    def sca(j, buf):
        pltpu.sync_copy(buf, agg_sh.at[cidx_v.at[j]], add=True)

    ng = jnp.where(c == 0, G0, G1)
    gbase = jnp.where(c == 0, s * G0, NTEC * G0 + s * G1)

    def group(g, carry):
        @pl.when(g < ng)
        def _():
            t = gbase + g
            pltpu.sync_copy(row_hbm.at[t], ridx_v)
            pltpu.sync_copy(col_hbm.at[t], cidx_v)
            gat_start(0, rows0_v, sem0)

            def pair(jj, inner):
                j0 = 2 * jj
                gat_start(j0 + 1, rows1_v, sem1)
                gat_wait(j0, rows0_v, sem0)
                sca(j0, rows0_v)

                @pl.when(jj < GCH // 2 - 1)
                def _():
                    gat_start(j0 + 2, rows0_v, sem0)

                gat_wait(j0 + 1, rows1_v, sem1)
                sca(j0 + 1, rows1_v)
                return inner

            lax.fori_loop(0, GCH // 2, pair, 0)
        return carry

    lax.fori_loop(0, max(G0, G1), group, 0)
    plsc.subcore_barrier()

    pltpu.sync_copy(agg_sh.at[pl.ds(base, ROWS_PER_TILE_C)],
                    out_hbm.at[c, pl.ds(base, ROWS_PER_TILE_C)])


_scatter_call = pl.kernel(
    _scatter_body,
    out_type=jax.ShapeDtypeStruct((NSC, NPAD, D), jnp.float32),
    mesh=_mesh,
    scratch_types=[
        pltpu.VMEM_SHARED((NPAD, D), jnp.float32),
        pltpu.VMEM((GCH, KC), jnp.int32),
        pltpu.VMEM((GCH, KC), jnp.int32),
        pltpu.VMEM((KC, D), jnp.float32),
        pltpu.VMEM((KC, D), jnp.float32),
        pltpu.SemaphoreType.DMA,
        pltpu.SemaphoreType.DMA,
    ],
)


def _prep_body(degp_ref, x_ref, y_ref):
    deg = degp_ref[0, :] + degp_ref[1, :] + 1.0
    dis = lax.rsqrt(deg)
    y_ref[...] = x_ref[...] * dis[:, None]


def _out_body(degp_ref, agg_ref, x_ref, x0_ref, w1_ref, w2_ref, o_ref):
    deg = degp_ref[0, :] + degp_ref[1, :] + 1.0
    dis = lax.rsqrt(deg)
    agg_raw = agg_ref[0] + agg_ref[1]
    agg = agg_raw * dis[:, None] + x_ref[...] * (1.0 / deg)[:, None]
    x0 = x0_ref[...]
    out = C1 * agg + BETA * jnp.dot(agg, w1_ref[...],
                                    preferred_element_type=jnp.float32)
    out = out + C2 * x0 + BETA * jnp.dot(x0, w2_ref[...],
                                         preferred_element_type=jnp.float32)
    o_ref[...] = jnp.maximum(out, 0.0)


_GB = 1024
_NB = NPAD // _GB

_prep_call = pl.pallas_call(
    _prep_body,
    grid=(_NB,),
    in_specs=[
        pl.BlockSpec((NSC, _GB), lambda i: (0, i)),
        pl.BlockSpec((_GB, D), lambda i: (i, 0)),
    ],
    out_specs=pl.BlockSpec((_GB, D), lambda i: (i, 0)),
    out_shape=jax.ShapeDtypeStruct((NPAD, D), jnp.float32),
)

_out_call = pl.pallas_call(
    _out_body,
    grid=(_NB,),
    in_specs=[
        pl.BlockSpec((NSC, _GB), lambda i: (0, i)),
        pl.BlockSpec((NSC, _GB, D), lambda i: (0, i, 0)),
        pl.BlockSpec((_GB, D), lambda i: (i, 0)),
        pl.BlockSpec((_GB, D), lambda i: (i, 0)),
        pl.BlockSpec((D, D), lambda i: (0, 0)),
        pl.BlockSpec((D, D), lambda i: (0, 0)),
    ],
    out_specs=pl.BlockSpec((_GB, D), lambda i: (i, 0)),
    out_shape=jax.ShapeDtypeStruct((NPAD, D), jnp.float32),
)


def kernel(x, x_0, edge_index, W1, W2):
    row = edge_index[0].astype(jnp.int32)
    col = edge_index[1].astype(jnp.int32)

    colA = jnp.pad(col, (0, NSC * NTEC * EPT_A_PAD - E),
                   constant_values=N).reshape(NSC * NTEC, NCHA, KA)

    epad = NSC * NTEC * EPT_C_PAD - E
    rowC = jnp.pad(row, (0, epad),
                   constant_values=0).reshape(NGRP, GCH, KC)
    colC = jnp.pad(col, (0, epad), constant_values=N + (NPAD - N) // 2)
    colC = colC.reshape(NGRP, GCH, KC)

    x_pad = jnp.pad(x, ((0, NPAD - N), (0, 0)))
    x0_pad = jnp.pad(x_0, ((0, NPAD - N), (0, 0)))

    degp = _deg_call(colA)
    y = _prep_call(degp, x_pad)
    agg2 = _scatter_call(y, rowC, colC)
    out = _out_call(degp, agg2, x_pad, x0_pad, W1, W2)
    return out[:N]


# speedup vs baseline: 1.4331x; 1.4331x over previous
import math

import jax
import jax.numpy as jnp
from jax import lax
from jax.experimental import pallas as pl
from jax.experimental.pallas import tpu as pltpu
from jax.experimental.pallas import tpu_sc as plsc

N = 10000
D = 128
E = 320000
NPAD = 10240
ALPHA = 0.5
BETA = math.log(2.0)
C1 = (1.0 - ALPHA) * (1.0 - BETA)
C2 = ALPHA * (1.0 - BETA)

NSC = 2
NTEC = 16

KA = 128
EPT_A = E // (NSC * NTEC)
NCHA = -(-EPT_A // KA)
EPT_A_PAD = NCHA * KA
ROWS_PER_TILE_A = NPAD // NTEC

KC = 64
GCH = 16
NGC = 10
NGRP = NSC * NTEC * NGC
EPT_C_PAD = NGC * GCH * KC
ROWS_PER_TILE_C = NPAD // NTEC
WW = D // 2

_mesh = plsc.VectorSubcoreMesh(core_axis_name="c", subcore_axis_name="s")


def _deg_body(col_hbm, out_hbm, hist_sh, idx_v, ones_v, zrow_v):
    c = lax.axis_index("c")
    s = lax.axis_index("s")
    t = c * NTEC + s
    base = s * ROWS_PER_TILE_A

    def fill_ones(i, carry):
        ones_v[pl.ds(i * 16, 16)] = jnp.ones((16,), jnp.float32)
        return carry

    lax.fori_loop(0, KA // 16, fill_ones, 0)

    def fill_zero(i, carry):
        zrow_v[pl.ds(i * 16, 16)] = jnp.zeros((16,), jnp.float32)
        return carry

    lax.fori_loop(0, ROWS_PER_TILE_A // 16, fill_zero, 0)

    pltpu.sync_copy(zrow_v, hist_sh.at[pl.ds(base, ROWS_PER_TILE_A)])
    pltpu.sync_copy(col_hbm.at[t], idx_v)
    plsc.subcore_barrier()

    def step(j, carry):
        pltpu.sync_copy(ones_v, hist_sh.at[idx_v.at[j]], add=True)
        return carry

    lax.fori_loop(0, NCHA, step, 0)
    plsc.subcore_barrier()
    pltpu.sync_copy(hist_sh.at[pl.ds(base, ROWS_PER_TILE_A)],
                    out_hbm.at[c, pl.ds(base, ROWS_PER_TILE_A)])


_deg_call = pl.kernel(
    _deg_body,
    out_type=jax.ShapeDtypeStruct((NSC, NPAD), jnp.float32),
    mesh=_mesh,
    scratch_types=[
        pltpu.VMEM_SHARED((NPAD,), jnp.float32),
        pltpu.VMEM((NCHA, KA), jnp.int32),
        pltpu.VMEM((KA,), jnp.float32),
        pltpu.VMEM((ROWS_PER_TILE_A,), jnp.float32),
    ],
)


def _scatter_body(y_hbm, row_hbm, col_hbm, out_hbm,
                  agg_sh, ridx_v, cidx_v, braw0_v, braw1_v, f0_v, f1_v,
                  gs0, gs1, ss0, ss1):
    c = lax.axis_index("c")
    s = lax.axis_index("s")
    base = s * ROWS_PER_TILE_C

    def zrow(r, carry):
        def zlane(k, inner):
            f0_v[r, pl.ds(k * 16, 16)] = jnp.zeros((16,), jnp.float32)
            return inner
        return lax.fori_loop(0, D // 16, zlane, carry)

    lax.fori_loop(0, KC, zrow, 0)

    def zcopy(k, carry):
        pltpu.sync_copy(f0_v, agg_sh.at[pl.ds(base + k * KC, KC)])
        return carry

    lax.fori_loop(0, ROWS_PER_TILE_C // KC, zcopy, 0)
    plsc.subcore_barrier()

    def gat_start(j, buf, sem):
        pltpu.async_copy(y_hbm.at[ridx_v.at[j]], buf, sem)

    def gat_wait(j, buf, sem):
        pltpu.make_async_copy(y_hbm.at[ridx_v.at[j]], buf, sem).wait()

    def sca_start(j, buf, sem):
        pltpu.async_copy(buf, agg_sh.at[cidx_v.at[j]], sem, add=True)

    def sca_wait(j, buf, sem):
        pltpu.make_async_copy(buf, agg_sh.at[cidx_v.at[j]], sem).wait()

    def convert(braw, fbuf):
        def conv(r, carry):
            for q in range(WW // 16):
                w = braw[r, pl.ds(q * 16, 16)]
                lo = plsc.bitcast(w << 16, jnp.float32)
                hi = plsc.bitcast(w & jnp.int32(-65536), jnp.float32)
                fbuf[r, pl.ds(q * 32, 16)] = lo
                fbuf[r, pl.ds(q * 32 + 16, 16)] = hi
            return carry
        lax.fori_loop(0, KC, conv, 0)

    t = c * NTEC + s

    def group(g, carry):
        tg = t * NGC + g
        pltpu.sync_copy(row_hbm.at[tg], ridx_v)
        pltpu.sync_copy(col_hbm.at[tg], cidx_v)
        gat_start(0, braw0_v, gs0)

        def pair(jj, inner):
            j0 = 2 * jj
            gat_start(j0 + 1, braw1_v, gs1)
            gat_wait(j0, braw0_v, gs0)

            @pl.when(jj > 0)
            def _():
                sca_wait(j0, f0_v, ss0)

            convert(braw0_v, f0_v)
            sca_start(j0, f0_v, ss0)

            @pl.when(jj < GCH // 2 - 1)
            def _():
                gat_start(j0 + 2, braw0_v, gs0)

            gat_wait(j0 + 1, braw1_v, gs1)

            @pl.when(jj > 0)
            def _():
                sca_wait(j0, f1_v, ss1)

            convert(braw1_v, f1_v)
            sca_start(j0 + 1, f1_v, ss1)
            return inner

        lax.fori_loop(0, GCH // 2, pair, 0)
        sca_wait(0, f0_v, ss0)
        sca_wait(0, f1_v, ss1)
        return carry

    lax.fori_loop(0, NGC, group, 0)
    plsc.subcore_barrier()

    pltpu.sync_copy(agg_sh.at[pl.ds(base, ROWS_PER_TILE_C)],
                    out_hbm.at[c, pl.ds(base, ROWS_PER_TILE_C)])


_scatter_call = pl.kernel(
    _scatter_body,
    out_type=jax.ShapeDtypeStruct((NSC, NPAD, D), jnp.float32),
    mesh=_mesh,
    compiler_params=pltpu.CompilerParams(use_tc_tiling_on_sc=False,
                                         needs_layout_passes=False),
    scratch_types=[
        pltpu.VMEM_SHARED((NPAD, D), jnp.float32),
        pltpu.VMEM((GCH, KC), jnp.int32),
        pltpu.VMEM((GCH, KC), jnp.int32),
        pltpu.VMEM((KC, WW), jnp.int32),
        pltpu.VMEM((KC, WW), jnp.int32),
        pltpu.VMEM((KC, D), jnp.float32),
        pltpu.VMEM((KC, D), jnp.float32),
        pltpu.SemaphoreType.DMA,
        pltpu.SemaphoreType.DMA,
        pltpu.SemaphoreType.DMA,
        pltpu.SemaphoreType.DMA,
    ],
)


def _prep_body(degp_ref, x_ref, y_ref):
    deg = degp_ref[0, :] + degp_ref[1, :] + 1.0
    dis = lax.rsqrt(deg)
    y_ref[...] = (x_ref[...] * dis[:, None]).astype(jnp.bfloat16)


def _out_body(degp_ref, agg_ref, x_ref, x0_ref, w1_ref, w2_ref, o_ref):
    deg = degp_ref[0, :] + degp_ref[1, :] + 1.0
    dis = lax.rsqrt(deg)
    agg_raw = agg_ref[0] + agg_ref[1]
    agg = agg_raw * dis[:, None] + x_ref[...] * (1.0 / deg)[:, None]
    x0 = x0_ref[...]
    out = C1 * agg + BETA * jnp.dot(agg, w1_ref[...],
                                    preferred_element_type=jnp.float32)
    out = out + C2 * x0 + BETA * jnp.dot(x0, w2_ref[...],
                                         preferred_element_type=jnp.float32)
    o_ref[...] = jnp.maximum(out, 0.0)


_GB = 1024
_NB = NPAD // _GB

_prep_call = pl.pallas_call(
    _prep_body,
    grid=(_NB,),
    in_specs=[
        pl.BlockSpec((NSC, _GB), lambda i: (0, i)),
        pl.BlockSpec((_GB, D), lambda i: (i, 0)),
    ],
    out_specs=pl.BlockSpec((_GB, D), lambda i: (i, 0)),
    out_shape=jax.ShapeDtypeStruct((NPAD, D), jnp.bfloat16),
)

_out_call = pl.pallas_call(
    _out_body,
    grid=(_NB,),
    in_specs=[
        pl.BlockSpec((NSC, _GB), lambda i: (0, i)),
        pl.BlockSpec((NSC, _GB, D), lambda i: (0, i, 0)),
        pl.BlockSpec((_GB, D), lambda i: (i, 0)),
        pl.BlockSpec((_GB, D), lambda i: (i, 0)),
        pl.BlockSpec((D, D), lambda i: (0, 0)),
        pl.BlockSpec((D, D), lambda i: (0, 0)),
    ],
    out_specs=pl.BlockSpec((_GB, D), lambda i: (i, 0)),
    out_shape=jax.ShapeDtypeStruct((NPAD, D), jnp.float32),
)


def kernel(x, x_0, edge_index, W1, W2):
    row = edge_index[0].astype(jnp.int32)
    col = edge_index[1].astype(jnp.int32)

    colA = jnp.pad(col, (0, NSC * NTEC * EPT_A_PAD - E),
                   constant_values=N).reshape(NSC * NTEC, NCHA, KA)

    epad = NSC * NTEC * EPT_C_PAD - E
    rowC = jnp.pad(row, (0, epad),
                   constant_values=0).reshape(NGRP, GCH, KC)
    colC = jnp.pad(col, (0, epad), constant_values=N + (NPAD - N) // 2)
    colC = colC.reshape(NGRP, GCH, KC)

    x_pad = jnp.pad(x, ((0, NPAD - N), (0, 0)))
    x0_pad = jnp.pad(x_0, ((0, NPAD - N), (0, 0)))

    degp = _deg_call(colA)
    yb = _prep_call(degp, x_pad)
    ybs = yb.reshape(NPAD, 4, 2, 16).transpose(0, 1, 3, 2)
    y_i32 = jax.lax.bitcast_convert_type(
        ybs.reshape(NPAD, WW, 2), jnp.int32)
    agg2 = _scatter_call(y_i32, rowC, colC)
    out = _out_call(degp, agg2, x_pad, x0_pad, W1, W2)
    return out[:N]
